# Initial kernel scaffold; baseline (speedup 1.0000x reference)
import jax
import jax.numpy as jnp
from jax.experimental import pallas as pl


def kernel(x, edge_index, W1, b1, W2, b2):
    raise NotImplementedError("write your pallas kernel here")


# speedup vs baseline: 48.2302x; 48.2302x over previous
import functools

import jax
import jax.numpy as jnp
from jax import lax
from jax.experimental import pallas as pl
from jax.experimental.pallas import tpu as pltpu
from jax.experimental.pallas import tpu_sc as plsc

NC = 2
NS = 16
NW = NC * NS
LANES = 16
WIN = 128

_mesh = plsc.VectorSubcoreMesh(
    core_axis_name="c", subcore_axis_name="s", num_cores=NC, num_subcores=NS
)


def _make_hist(NP, NWIN):
    rows = NP // NS

    @functools.partial(
        pl.kernel,
        out_type=jax.ShapeDtypeStruct((NC, NP), jnp.float32),
        mesh=_mesh,
        scratch_types=[
            pltpu.VMEM((NWIN, WIN), jnp.int32),
            pltpu.VMEM((WIN,), jnp.float32),
            pltpu.VMEM((rows,), jnp.float32),
            pltpu.VMEM_SHARED((NP,), jnp.float32),
        ],
    )
    def hist(dstw, out, idx_v, ones_v, z_v, acc):
        c = lax.axis_index("c")
        s = lax.axis_index("s")
        wid = c * NS + s

        def fill_ones(i, carry):
            ones_v[pl.ds(i * LANES, LANES)] = jnp.ones((LANES,), jnp.float32)
            return carry

        lax.fori_loop(0, WIN // LANES, fill_ones, 0)

        def fill_zero(i, carry):
            z_v[pl.ds(i * LANES, LANES)] = jnp.zeros((LANES,), jnp.float32)
            return carry

        lax.fori_loop(0, rows // LANES, fill_zero, 0)
        pltpu.sync_copy(z_v, acc.at[pl.ds(s * rows, rows)])
        pltpu.sync_copy(dstw.at[wid], idx_v)
        plsc.subcore_barrier()

        def body(j, carry):
            pltpu.sync_copy(ones_v, acc.at[idx_v.at[j]], add=True)
            return carry

        lax.fori_loop(0, NWIN, body, 0)
        plsc.subcore_barrier()
        pltpu.sync_copy(
            acc.at[pl.ds(s * rows, rows)], out.at[c, pl.ds(s * rows, rows)]
        )

    return hist


def _make_edge_wide(NP, NWIN, F):
    rows = NP // NS

    @functools.partial(
        pl.kernel,
        out_type=jax.ShapeDtypeStruct((NC, NP, F), jnp.float32),
        mesh=_mesh,
        compiler_params=pltpu.CompilerParams(use_tc_tiling_on_sc=False),
        scratch_types=[
            pltpu.VMEM((NWIN, WIN), jnp.int32),
            pltpu.VMEM((NWIN, WIN), jnp.int32),
            pltpu.VMEM((WIN, F), jnp.float32),
            pltpu.VMEM((rows, F), jnp.float32),
            pltpu.VMEM_SHARED((NP, F), jnp.float32),
            pltpu.SemaphoreType.DMA,
        ],
    )
    def edge(srcw, dstw, g, out, src_v, dst_v, gbuf, z_v, acc, sem):
        c = lax.axis_index("c")
        s = lax.axis_index("s")
        wid = c * NS + s

        def fill_zero(i, carry):
            z_v[i] = jnp.zeros((F,), jnp.float32)
            return carry

        lax.fori_loop(0, rows, fill_zero, 0)
        pltpu.sync_copy(z_v, acc.at[pl.ds(s * rows, rows)])
        pltpu.sync_copy(srcw.at[wid], src_v)
        pltpu.sync_copy(dstw.at[wid], dst_v)
        plsc.subcore_barrier()

        def body(j, carry):
            pltpu.async_copy(g.at[src_v.at[j]], gbuf, sem).wait()
            pltpu.sync_copy(gbuf, acc.at[dst_v.at[j]], add=True)
            return carry

        lax.fori_loop(0, NWIN, body, 0)
        plsc.subcore_barrier()
        pltpu.sync_copy(
            acc.at[pl.ds(s * rows, rows)], out.at[c, pl.ds(s * rows, rows)]
        )

    return edge


def _make_edge_scalar(NP, NWIN):
    rows = NP // NS

    @functools.partial(
        pl.kernel,
        out_type=jax.ShapeDtypeStruct((NC, NP), jnp.float32),
        mesh=_mesh,
        scratch_types=[
            pltpu.VMEM((NWIN, WIN), jnp.int32),
            pltpu.VMEM((NWIN, WIN), jnp.int32),
            pltpu.VMEM((WIN,), jnp.float32),
            pltpu.VMEM((rows,), jnp.float32),
            pltpu.VMEM_SHARED((NP,), jnp.float32),
            pltpu.VMEM_SHARED((NP,), jnp.float32),
            pltpu.SemaphoreType.DMA,
        ],
    )
    def edge(srcw, dstw, g, out, src_v, dst_v, gbuf, z_v, acc, g_sh, sem):
        c = lax.axis_index("c")
        s = lax.axis_index("s")
        wid = c * NS + s

        def fill_zero(i, carry):
            z_v[pl.ds(i * LANES, LANES)] = jnp.zeros((LANES,), jnp.float32)
            return carry

        lax.fori_loop(0, rows // LANES, fill_zero, 0)
        pltpu.sync_copy(z_v, acc.at[pl.ds(s * rows, rows)])
        pltpu.sync_copy(
            g.at[pl.ds(s * rows, rows)], g_sh.at[pl.ds(s * rows, rows)]
        )
        pltpu.sync_copy(srcw.at[wid], src_v)
        pltpu.sync_copy(dstw.at[wid], dst_v)
        plsc.subcore_barrier()

        def body(j, carry):
            pltpu.async_copy(g_sh.at[src_v.at[j]], gbuf, sem).wait()
            pltpu.sync_copy(gbuf, acc.at[dst_v.at[j]], add=True)
            return carry

        lax.fori_loop(0, NWIN, body, 0)
        plsc.subcore_barrier()
        pltpu.sync_copy(
            acc.at[pl.ds(s * rows, rows)], out.at[c, pl.ds(s * rows, rows)]
        )

    return edge


def _make_tca(NP, N, H):
    def body(xp, w1, hist_t, g1_out, dinv_out):
        deg = hist_t[:, 0:1] + hist_t[:, 1:2] + 1.0
        iota = lax.broadcasted_iota(jnp.int32, (NP, 1), 0)
        dinv = jnp.where(iota < N, lax.rsqrt(deg), 0.0)
        h1 = jnp.dot(xp[...], w1[...], preferred_element_type=jnp.float32)
        g1_out[...] = h1 * dinv
        dinv_out[...] = dinv

    return pl.pallas_call(
        body,
        out_shape=(
            jax.ShapeDtypeStruct((NP, H), jnp.float32),
            jax.ShapeDtypeStruct((NP, 1), jnp.float32),
        ),
    )


def _make_tcb(NP):
    def body(a0, a1, g1, dinv, b1r, w2, g2_out):
        s1 = a0[...] + a1[...] + g1[...]
        out1 = s1 * dinv[...] + b1r[...]
        r = jnp.maximum(out1, 0.0)
        h2 = jnp.dot(r, w2[...], preferred_element_type=jnp.float32)
        g2_out[...] = h2 * dinv[...]

    return pl.pallas_call(
        body, out_shape=jax.ShapeDtypeStruct((NP, 1), jnp.float32)
    )


def _make_tcc(NP):
    def body(agg2_t, g2, dinv, b2r, out):
        a = agg2_t[:, 0:1] + agg2_t[:, 1:2]
        z = (a + g2[...]) * dinv[...] + b2r[...]
        out[...] = jax.nn.sigmoid(z)

    return pl.pallas_call(
        body, out_shape=jax.ShapeDtypeStruct((NP, 1), jnp.float32)
    )


def kernel(x, edge_index, W1, b1, W2, b2):
    N, D = x.shape
    H = W1.shape[1]
    E = edge_index.shape[1]

    blk = NS * LANES * NC
    NP = (N // blk + 1) * blk
    EW = -(-E // NW)
    NWIN = -(-EW // WIN)
    total = NW * NWIN * WIN

    src = edge_index[0].astype(jnp.int32)
    dst = edge_index[1].astype(jnp.int32)
    padn = total - E
    padidx = N + (jnp.arange(padn, dtype=jnp.int32) % (NP - N))
    srcp = jnp.concatenate([src, padidx]).reshape(NW, NWIN, WIN)
    dstp = jnp.concatenate([dst, padidx]).reshape(NW, NWIN, WIN)
    x_p = jnp.pad(x, ((0, NP - N), (0, 0)))
    b1r = b1.reshape(1, H)
    b2r = b2.reshape(1, 1)

    hist = _make_hist(NP, NWIN)(dstp)
    g1, dinv = _make_tca(NP, N, H)(x_p, W1, hist.T)
    agg1 = _make_edge_wide(NP, NWIN, H)(srcp, dstp, g1)
    g2 = _make_tcb(NP)(agg1[0], agg1[1], g1, dinv, b1r, W2)
    agg2 = _make_edge_scalar(NP, NWIN)(srcp, dstp, g2.reshape(NP))
    out = _make_tcc(NP)(agg2.T, g2, dinv, b2r)
    return out[:N]


# speedup vs baseline: 56.6514x; 1.1746x over previous
import functools

import jax
import jax.numpy as jnp
from jax import lax
from jax.experimental import pallas as pl
from jax.experimental.pallas import tpu as pltpu
from jax.experimental.pallas import tpu_sc as plsc

NC = 2
NS = 16
NW = NC * NS
LANES = 16
WIN = 128
K = 4

_mesh = plsc.VectorSubcoreMesh(
    core_axis_name="c", subcore_axis_name="s", num_cores=NC, num_subcores=NS
)
_params = pltpu.CompilerParams(use_tc_tiling_on_sc=False)


def _make_hist(NP, NWIN):
    rows = NP // NS
    G = 8
    assert NWIN % G == 0

    @functools.partial(
        pl.kernel,
        out_type=jax.ShapeDtypeStruct((NC, NP), jnp.float32),
        mesh=_mesh,
        compiler_params=_params,
        scratch_types=[
            pltpu.VMEM((NWIN, WIN), jnp.int32),
            pltpu.VMEM((WIN,), jnp.float32),
            pltpu.VMEM((rows,), jnp.float32),
            pltpu.VMEM_SHARED((NP,), jnp.float32),
            pltpu.SemaphoreType.DMA,
        ],
    )
    def hist(dstw, out, idx_v, ones_v, z_v, acc, sem):
        c = lax.axis_index("c")
        s = lax.axis_index("s")
        wid = c * NS + s

        def fill_ones(i, carry):
            ones_v[pl.ds(i * LANES, LANES)] = jnp.ones((LANES,), jnp.float32)
            return carry

        lax.fori_loop(0, WIN // LANES, fill_ones, 0)

        def fill_zero(i, carry):
            z_v[pl.ds(i * LANES, LANES)] = jnp.zeros((LANES,), jnp.float32)
            return carry

        lax.fori_loop(0, rows // LANES, fill_zero, 0)
        pltpu.sync_copy(z_v, acc.at[pl.ds(s * rows, rows)])
        pltpu.sync_copy(dstw.at[wid], idx_v)
        plsc.subcore_barrier()

        def body(gi, carry):
            for b in range(G):
                pltpu.async_copy(
                    ones_v, acc.at[idx_v.at[gi * G + b]], sem, add=True
                )
            for b in range(G):
                pltpu.make_async_copy(
                    ones_v, acc.at[idx_v.at[gi * G + b]], sem
                ).wait()
            return carry

        lax.fori_loop(0, NWIN // G, body, 0)
        plsc.subcore_barrier()
        pltpu.sync_copy(
            acc.at[pl.ds(s * rows, rows)], out.at[c, pl.ds(s * rows, rows)]
        )

    return hist


def _make_edge(NP, NWIN, F):
    rows = NP // NS
    NG2 = NWIN // (2 * K)
    assert NWIN == NG2 * 2 * K
    vshape = (WIN, F) if F else (WIN,)
    bufshape = (K,) + vshape
    accshape = (NP, F) if F else (NP,)
    zshape = (rows, F) if F else (rows,)
    outshape = (NC, NP, F) if F else (NC, NP)

    @functools.partial(
        pl.kernel,
        out_type=jax.ShapeDtypeStruct(outshape, jnp.float32),
        mesh=_mesh,
        compiler_params=_params,
        scratch_types=[
            pltpu.VMEM((NWIN, WIN), jnp.int32),
            pltpu.VMEM((NWIN, WIN), jnp.int32),
            pltpu.VMEM(bufshape, jnp.float32),
            pltpu.VMEM(bufshape, jnp.float32),
            pltpu.VMEM(zshape, jnp.float32),
            pltpu.VMEM_SHARED(accshape, jnp.float32),
            pltpu.SemaphoreType.DMA,
            pltpu.SemaphoreType.DMA,
            pltpu.SemaphoreType.DMA,
            pltpu.SemaphoreType.DMA,
        ],
    )
    def edge(srcw, dstw, g, out, src_v, dst_v, buf0, buf1, z_v, acc,
             semg0, semg1, sems0, sems1):
        c = lax.axis_index("c")
        s = lax.axis_index("s")
        wid = c * NS + s

        if F:
            def fill_zero(i, carry):
                z_v[i] = jnp.zeros((F,), jnp.float32)
                return carry

            lax.fori_loop(0, rows, fill_zero, 0)
        else:
            def fill_zero(i, carry):
                z_v[pl.ds(i * LANES, LANES)] = jnp.zeros(
                    (LANES,), jnp.float32
                )
                return carry

            lax.fori_loop(0, rows // LANES, fill_zero, 0)
        pltpu.sync_copy(z_v, acc.at[pl.ds(s * rows, rows)])
        pltpu.sync_copy(srcw.at[wid], src_v)
        pltpu.sync_copy(dstw.at[wid], dst_v)
        plsc.subcore_barrier()

        def gath(j, buf, b, sem):
            pltpu.async_copy(g.at[src_v.at[j]], buf.at[b], sem)

        def gath_wait(buf, b, sem):
            pltpu.make_async_copy(g.at[src_v.at[0]], buf.at[b], sem).wait()

        def scat(j, buf, b, sem):
            pltpu.async_copy(buf.at[b], acc.at[dst_v.at[j]], sem, add=True)

        def scat_wait(buf, b, sem):
            pltpu.make_async_copy(
                buf.at[b], acc.at[dst_v.at[0]], sem
            ).wait()

        for b in range(K):
            gath(b, buf0, b, semg0)

        def pair(p, carry):
            j0 = 2 * p * K
            j1 = j0 + K

            @pl.when(p > 0)
            def _():
                for b in range(K):
                    scat_wait(buf1, b, sems1)

            for b in range(K):
                gath(j1 + b, buf1, b, semg1)
            for b in range(K):
                gath_wait(buf0, b, semg0)
            for b in range(K):
                scat(j0 + b, buf0, b, sems0)
            for b in range(K):
                scat_wait(buf0, b, sems0)

            @pl.when(p < NG2 - 1)
            def _():
                for b in range(K):
                    gath(j1 + K + b, buf0, b, semg0)

            for b in range(K):
                gath_wait(buf1, b, semg1)
            for b in range(K):
                scat(j1 + b, buf1, b, sems1)
            return carry

        lax.fori_loop(0, NG2, pair, 0)
        for b in range(K):
            scat_wait(buf1, b, sems1)
        plsc.subcore_barrier()
        pltpu.sync_copy(
            acc.at[pl.ds(s * rows, rows)], out.at[c, pl.ds(s * rows, rows)]
        )

    return edge


def _make_tca(NP, N, H):
    def body(xp, w1, hist_t, g1_out, dinv_out):
        deg = hist_t[:, 0:1] + hist_t[:, 1:2] + 1.0
        iota = lax.broadcasted_iota(jnp.int32, (NP, 1), 0)
        dinv = jnp.where(iota < N, lax.rsqrt(deg), 0.0)
        h1 = jnp.dot(xp[...], w1[...], preferred_element_type=jnp.float32)
        g1_out[...] = h1 * dinv
        dinv_out[...] = dinv

    return pl.pallas_call(
        body,
        out_shape=(
            jax.ShapeDtypeStruct((NP, H), jnp.float32),
            jax.ShapeDtypeStruct((NP, 1), jnp.float32),
        ),
    )


def _make_tcb(NP):
    def body(a0, a1, g1, dinv, b1r, w2, g2_out):
        s1 = a0[...] + a1[...] + g1[...]
        out1 = s1 * dinv[...] + b1r[...]
        r = jnp.maximum(out1, 0.0)
        h2 = jnp.dot(r, w2[...], preferred_element_type=jnp.float32)
        g2_out[...] = h2 * dinv[...]

    return pl.pallas_call(
        body, out_shape=jax.ShapeDtypeStruct((NP, 1), jnp.float32)
    )


def _make_tcc(NP):
    def body(agg2_t, g2, dinv, b2r, out):
        a = agg2_t[:, 0:1] + agg2_t[:, 1:2]
        z = (a + g2[...]) * dinv[...] + b2r[...]
        out[...] = jax.nn.sigmoid(z)

    return pl.pallas_call(
        body, out_shape=jax.ShapeDtypeStruct((NP, 1), jnp.float32)
    )


def kernel(x, edge_index, W1, b1, W2, b2):
    N, D = x.shape
    H = W1.shape[1]
    E = edge_index.shape[1]

    blk = NS * LANES * NC
    NP = (N // blk + 1) * blk
    EW = -(-E // NW)
    NWIN = -(-EW // WIN)
    NWIN = -(-NWIN // (2 * K)) * (2 * K)
    total = NW * NWIN * WIN

    src = edge_index[0].astype(jnp.int32)
    dst = edge_index[1].astype(jnp.int32)
    padn = total - E
    padidx = N + (jnp.arange(padn, dtype=jnp.int32) % (NP - N))
    srcp = jnp.concatenate([src, padidx]).reshape(NW, NWIN, WIN)
    dstp = jnp.concatenate([dst, padidx]).reshape(NW, NWIN, WIN)
    x_p = jnp.pad(x, ((0, NP - N), (0, 0)))
    b1r = b1.reshape(1, H)
    b2r = b2.reshape(1, 1)

    hist = _make_hist(NP, NWIN)(dstp)
    g1, dinv = _make_tca(NP, N, H)(x_p, W1, hist.T)
    agg1 = _make_edge(NP, NWIN, H)(srcp, dstp, g1)
    g2 = _make_tcb(NP)(agg1[0], agg1[1], g1, dinv, b1r, W2)
    agg2 = _make_edge(NP, NWIN, 0)(srcp, dstp, g2.reshape(NP))
    out = _make_tcc(NP)(agg2.T, g2, dinv, b2r)
    return out[:N]
